# baseline (speedup 1.0000x reference)
import functools

import jax
import jax.numpy as jnp
from jax import lax
from jax.experimental import pallas as pl
from jax.experimental.pallas import tpu as pltpu
from jax.experimental.pallas import tpu_sc as plsc

B = 16
L = 4096
P = 1000
NWORKERS = 32
SUPER = 128
CHUNK = 32
NSUB = SUPER // CHUNK
NFULL = P // 16
TAIL_OFF = P - 16


def _sc_partials_kernel(x_hbm, n_hbm, table_hbm, out_hbm,
                        n_stage, idx_v, rows_v, acc_v, sem):
    wid = lax.axis_index("s") * 2 + lax.axis_index("c")
    zeros16 = jnp.zeros((16,), jnp.float32)
    iota16 = lax.iota(jnp.int32, 16)
    tail_mask = iota16 >= 8

    for b in range(B):
        def _zero_body(k, _):
            acc_v[b, pl.ds(k * 16, 16)] = zeros16
            return 0
        lax.fori_loop(0, NFULL, _zero_body, 0)
        acc_v[b, pl.ds(TAIL_OFF, 16)] = zeros16

    pltpu.sync_copy(n_hbm, n_stage)
    n_vec = n_stage[...]

    for b in range(B):
        nb = jnp.sum(jnp.where(iota16 == b, n_vec, 0))
        pos = lax.rem(wid + 17 * b, NWORKERS)

        def _chunk_body(c, _, b=b, nb=nb, pos=pos):
            l0 = pos * SUPER + c * CHUNK
            v = jnp.clip(nb - l0, 0, CHUNK)

            @pl.when(v > 0)
            def _():
                pltpu.sync_copy(x_hbm.at[b, pl.ds(l0, CHUNK)], idx_v)
                pltpu.async_copy(table_hbm.at[idx_v], rows_v, sem).wait()

                def _row_body(r, _):
                    for k in range(NFULL):
                        plsc.addupdate(acc_v.at[b, pl.ds(k * 16, 16)],
                                       rows_v[r, pl.ds(k * 16, 16)])
                    t = rows_v[r, pl.ds(TAIL_OFF, 16)]
                    plsc.addupdate(acc_v.at[b, pl.ds(TAIL_OFF, 16)],
                                   jnp.where(tail_mask, t, 0.0))
                    return 0
                lax.fori_loop(0, v, _row_body, 0)
            return 0

        lax.fori_loop(0, NSUB, _chunk_body, 0)

    pltpu.sync_copy(acc_v, out_hbm.at[wid])


_sc_partials = functools.partial(
    pl.kernel,
    out_type=jax.ShapeDtypeStruct((NWORKERS, B, P), jnp.float32),
    mesh=plsc.VectorSubcoreMesh(core_axis_name="c", subcore_axis_name="s"),
    scratch_types=[
        pltpu.VMEM((16,), jnp.int32),
        pltpu.VMEM((CHUNK,), jnp.int32),
        pltpu.VMEM((CHUNK, P), jnp.float32),
        pltpu.VMEM((B, P), jnp.float32),
        pltpu.SemaphoreType.DMA,
    ],
    compiler_params=pltpu.CompilerParams(use_tc_tiling_on_sc=False,
                                         needs_layout_passes=False),
)(_sc_partials_kernel)


def _tc_finish_kernel(partials_ref, n_ref, gamma_ref, beta_ref, w_ref,
                      b_ref, out_ref):
    s = jnp.sum(partials_ref[...], axis=0)
    nf = n_ref[...].astype(jnp.float32)
    x = s / nf[:, None]
    mean = jnp.mean(x, axis=-1, keepdims=True)
    var = jnp.mean((x - mean) ** 2, axis=-1, keepdims=True)
    xn = (x - mean) * lax.rsqrt(var + 1e-5)
    xn = xn * gamma_ref[...][None, :] + beta_ref[...][None, :]
    logits = jnp.dot(xn, w_ref[...],
                     preferred_element_type=jnp.float32) + b_ref[...][None, :]
    out_ref[...] = jax.nn.sigmoid(logits)[:, 0]


_tc_finish = pl.pallas_call(
    _tc_finish_kernel,
    out_shape=jax.ShapeDtypeStruct((B,), jnp.float32),
)


def kernel(X, N, table, gamma, beta, W, b):
    X = X.astype(jnp.int32)
    N = N.astype(jnp.int32)
    partials = _sc_partials(X, N, table)
    return _tc_finish(partials, N, gamma, beta, W, b)


# speedup vs baseline: 4.4050x; 4.4050x over previous
import functools

import jax
import jax.numpy as jnp
from jax import lax
from jax.experimental import pallas as pl
from jax.experimental.pallas import tpu as pltpu
from jax.experimental.pallas import tpu_sc as plsc

B = 16
L = 4096
P = 1000
NWORKERS = 32
SUPER = 128
CHUNK = 32
NSUB = SUPER // CHUNK
NFULL = P // 16
TAIL_OFF = P - 16


def _sc_partials_kernel(x_hbm, n_hbm, table_hbm, out_hbm,
                        n_stage, idx_v, rows_v, acc_v, sem):
    wid = lax.axis_index("s") * 2 + lax.axis_index("c")
    zeros16 = jnp.zeros((16,), jnp.float32)
    iota16 = lax.iota(jnp.int32, 16)
    tail_mask = iota16 >= 8

    pltpu.sync_copy(n_hbm, n_stage)
    n_vec = n_stage[...]

    def _batch_body(b, _):
        nb = jnp.sum(jnp.where(iota16 == b, n_vec, 0))
        pos = lax.rem(wid + 17 * b, NWORKERS)

        def _zero_body(k, _):
            acc_v[b, pl.ds(k * 16, 16)] = zeros16
            return 0
        lax.fori_loop(0, NFULL, _zero_body, 0)
        acc_v[b, pl.ds(TAIL_OFF, 16)] = zeros16

        def _chunk_body(c, _):
            l0 = pos * SUPER + c * CHUNK
            v = jnp.clip(nb - l0, 0, CHUNK)

            @pl.when(v > 0)
            def _():
                pltpu.async_copy(x_hbm.at[b, pl.ds(l0, CHUNK)], idx_v,
                                 sem).wait()

                def _fire(r, _):
                    lane = lax.rem(r, 16)
                    vec = idx_v[pl.ds(lax.div(r, 16) * 16, 16)]
                    ridx = jnp.sum(jnp.where(iota16 == lane, vec, 0))
                    pltpu.async_copy(table_hbm.at[ridx], rows_v.at[r],
                                     sem)
                    return 0
                lax.fori_loop(0, v, _fire, 0)

                def _drain(r, _):
                    pltpu.make_async_copy(table_hbm.at[0], rows_v.at[0],
                                          sem).wait()
                    return 0
                lax.fori_loop(0, v, _drain, 0)

                def _row_body(r, _):
                    for k in range(NFULL):
                        plsc.addupdate(acc_v.at[b, pl.ds(k * 16, 16)],
                                       rows_v[r, pl.ds(k * 16, 16)])
                    t = rows_v[r, pl.ds(TAIL_OFF, 16)]
                    plsc.addupdate(acc_v.at[b, pl.ds(TAIL_OFF, 16)],
                                   jnp.where(tail_mask, t, 0.0))
                    return 0
                lax.fori_loop(0, v, _row_body, 0)
            return 0

        lax.fori_loop(0, NSUB, _chunk_body, 0)
        return 0

    lax.fori_loop(0, B, _batch_body, 0)

    pltpu.sync_copy(acc_v, out_hbm.at[wid])


_sc_partials = functools.partial(
    pl.kernel,
    out_type=jax.ShapeDtypeStruct((NWORKERS, B, P), jnp.float32),
    mesh=plsc.VectorSubcoreMesh(core_axis_name="c", subcore_axis_name="s"),
    scratch_types=[
        pltpu.VMEM((16,), jnp.int32),
        pltpu.VMEM((CHUNK,), jnp.int32),
        pltpu.VMEM((CHUNK, P), jnp.float32),
        pltpu.VMEM((B, P), jnp.float32),
        pltpu.SemaphoreType.DMA,
    ],
    compiler_params=pltpu.CompilerParams(needs_layout_passes=False),
)(_sc_partials_kernel)


def _tc_finish_kernel(partials_ref, n_ref, gamma_ref, beta_ref, w_ref,
                      b_ref, out_ref):
    s = jnp.sum(partials_ref[...], axis=0)
    nf = n_ref[...].astype(jnp.float32)
    x = s / nf[:, None]
    mean = jnp.mean(x, axis=-1, keepdims=True)
    var = jnp.mean((x - mean) ** 2, axis=-1, keepdims=True)
    xn = (x - mean) * lax.rsqrt(var + 1e-5)
    xn = xn * gamma_ref[...][None, :] + beta_ref[...][None, :]
    logits = jnp.dot(xn, w_ref[...],
                     preferred_element_type=jnp.float32) + b_ref[...][None, :]
    out_ref[...] = jax.nn.sigmoid(logits)[:, 0]


_tc_finish = pl.pallas_call(
    _tc_finish_kernel,
    out_shape=jax.ShapeDtypeStruct((B,), jnp.float32),
)


def kernel(X, N, table, gamma, beta, W, b):
    X = X.astype(jnp.int32)
    N = N.astype(jnp.int32)
    partials = _sc_partials(X, N, table)
    return _tc_finish(partials, N, gamma, beta, W, b)


# speedup vs baseline: 14.2017x; 3.2240x over previous
import functools

import jax
import jax.numpy as jnp
from jax import lax
from jax.experimental import pallas as pl
from jax.experimental.pallas import tpu as pltpu
from jax.experimental.pallas import tpu_sc as plsc

B = 16
L = 4096
P = 1000
VOCAB = 200019
KB = 2048
NSTEPS = -(-VOCAB // KB)
KPAD = NSTEPS * KB
NWORKERS = 32
VR = KPAD // NWORKERS


def _sc_hist_kernel(x_hbm, n_hbm, c_hbm, n_stage, xbuf, tile_c, sem):
    wid = lax.axis_index("s") * 2 + lax.axis_index("c")
    v0 = wid * VR
    zeros16 = jnp.zeros((16,), jnp.float32)
    ones16 = jnp.ones((16,), jnp.float32)
    iota16 = lax.iota(jnp.int32, 16)

    for b in range(B):
        def _zb(k, _, b=b):
            tile_c[b, pl.ds(k * 16, 16)] = zeros16
            return 0
        lax.fori_loop(0, VR // 16, _zb, 0)

    pltpu.sync_copy(n_hbm, n_stage)
    n_vec = n_stage[...]

    for b in range(B):
        nb = jnp.sum(jnp.where(iota16 == b, n_vec, 0))
        pltpu.async_copy(x_hbm.at[b], xbuf, sem).wait()

        def _vreg(j, _, b=b, nb=nb):
            xv = xbuf[pl.ds(j * 16, 16)]
            local = xv - v0
            mask = ((local >= 0) & (local < VR)
                    & (j * 16 + iota16 < nb))
            row = jnp.full((16,), b, jnp.int32)
            plsc.addupdate_scatter(tile_c, [row, local], ones16,
                                   mask=mask)
            return 0

        lax.fori_loop(0, (nb + 15) // 16, _vreg, 0)

    pltpu.sync_copy(tile_c, c_hbm.at[:, pl.ds(v0, VR)])


_sc_hist = functools.partial(
    pl.kernel,
    out_type=jax.ShapeDtypeStruct((B, KPAD), jnp.float32),
    mesh=plsc.VectorSubcoreMesh(core_axis_name="c", subcore_axis_name="s"),
    scratch_types=[
        pltpu.VMEM((16,), jnp.int32),
        pltpu.VMEM((L,), jnp.int32),
        pltpu.VMEM((B, VR), jnp.float32),
        pltpu.SemaphoreType.DMA,
    ],
    compiler_params=pltpu.CompilerParams(needs_layout_passes=False),
)(_sc_hist_kernel)


def _tc_matmul_kernel(c_ref, t_ref, n_ref, gamma_ref, beta_ref, w_ref,
                      b_ref, out_ref, acc_ref):
    k = pl.program_id(0)

    @pl.when(k == 0)
    def _():
        acc_ref[...] = jnp.zeros_like(acc_ref)

    acc_ref[...] += lax.dot_general(
        c_ref[...], t_ref[...],
        dimension_numbers=(((1,), (1,)), ((), ())),
        preferred_element_type=jnp.float32)

    @pl.when(k == NSTEPS - 1)
    def _():
        s = acc_ref[...]
        nf = n_ref[...].astype(jnp.float32)
        x = s / nf[:, None]
        mean = jnp.mean(x, axis=-1, keepdims=True)
        var = jnp.mean((x - mean) ** 2, axis=-1, keepdims=True)
        xn = (x - mean) * lax.rsqrt(var + 1e-5)
        xn = xn * gamma_ref[...][None, :] + beta_ref[...][None, :]
        logits = jnp.dot(xn, w_ref[...],
                         preferred_element_type=jnp.float32)
        logits = logits + b_ref[...][None, :]
        out_ref[...] = jax.nn.sigmoid(logits)[:, 0]


_tc_matmul = pl.pallas_call(
    _tc_matmul_kernel,
    grid=(NSTEPS,),
    in_specs=[
        pl.BlockSpec((B, KB), lambda k: (0, k)),
        pl.BlockSpec((P, KB), lambda k: (0, k)),
        pl.BlockSpec((B,), lambda k: (0,)),
        pl.BlockSpec((P,), lambda k: (0,)),
        pl.BlockSpec((P,), lambda k: (0,)),
        pl.BlockSpec((P, 1), lambda k: (0, 0)),
        pl.BlockSpec((1,), lambda k: (0,)),
    ],
    out_specs=pl.BlockSpec((B,), lambda k: (0,)),
    out_shape=jax.ShapeDtypeStruct((B,), jnp.float32),
    scratch_shapes=[pltpu.VMEM((B, P), jnp.float32)],
)


def kernel(X, N, table, gamma, beta, W, b):
    X = X.astype(jnp.int32)
    N = N.astype(jnp.int32)
    counts = _sc_hist(X, N)
    table_t = jnp.swapaxes(table, 0, 1)
    return _tc_matmul(counts, table_t, N, gamma, beta, W, b)


# speedup vs baseline: 15.5064x; 1.0919x over previous
import functools

import jax
import jax.numpy as jnp
from jax import lax
from jax.experimental import pallas as pl
from jax.experimental.pallas import tpu as pltpu
from jax.experimental.pallas import tpu_sc as plsc

B = 16
L = 4096
P = 1000
VOCAB = 200019
KB = 2048
NSTEPS = -(-VOCAB // KB)
KPAD = NSTEPS * KB
NWORKERS = 32
VR = KPAD // NWORKERS


def _sc_hist_kernel(x_hbm, n_hbm, c_hbm, n_stage, xbuf0, xbuf1, tile_c,
                    sem):
    wid = lax.axis_index("s") * 2 + lax.axis_index("c")
    v0 = wid * VR
    zeros16 = jnp.zeros((16,), jnp.float32)
    ones16 = jnp.ones((16,), jnp.float32)
    iota16 = lax.iota(jnp.int32, 16)
    xbufs = [xbuf0, xbuf1]

    for b in range(B):
        def _zb(k, _, b=b):
            for u in range(8):
                tile_c[b, pl.ds(k * 128 + u * 16, 16)] = zeros16
            return 0
        lax.fori_loop(0, VR // 128, _zb, 0)

    pltpu.sync_copy(n_hbm, n_stage)
    n_vec = n_stage[...]
    pltpu.async_copy(x_hbm.at[0], xbuf0, sem)

    for b in range(B):
        nb = jnp.sum(jnp.where(iota16 == b, n_vec, 0))
        xbuf = xbufs[b % 2]
        pltpu.make_async_copy(x_hbm.at[b], xbuf, sem).wait()
        if b + 1 < B:
            pltpu.async_copy(x_hbm.at[b + 1], xbufs[(b + 1) % 2], sem)

        def _vreg(j, _, b=b, nb=nb, xbuf=xbuf):
            row = jnp.full((16,), b, jnp.int32)
            for u in range(4):
                xv = xbuf[pl.ds(j * 64 + u * 16, 16)]
                local = xv - v0
                mask = ((local >= 0) & (local < VR)
                        & (j * 64 + u * 16 + iota16 < nb))
                plsc.addupdate_scatter(tile_c, [row, local], ones16,
                                       mask=mask)
            return 0

        lax.fori_loop(0, (nb + 63) // 64, _vreg, 0)

    pltpu.sync_copy(tile_c, c_hbm.at[:, pl.ds(v0, VR)])


_sc_hist = functools.partial(
    pl.kernel,
    out_type=jax.ShapeDtypeStruct((B, KPAD), jnp.float32),
    mesh=plsc.VectorSubcoreMesh(core_axis_name="c", subcore_axis_name="s"),
    scratch_types=[
        pltpu.VMEM((16,), jnp.int32),
        pltpu.VMEM((L,), jnp.int32),
        pltpu.VMEM((L,), jnp.int32),
        pltpu.VMEM((B, VR), jnp.float32),
        pltpu.SemaphoreType.DMA,
    ],
    compiler_params=pltpu.CompilerParams(needs_layout_passes=False),
)(_sc_hist_kernel)


def _tc_matmul_kernel(c_ref, t_ref, n_ref, gamma_ref, beta_ref, w_ref,
                      b_ref, out_ref, acc_ref):
    k = pl.program_id(0)

    @pl.when(k == 0)
    def _():
        acc_ref[...] = jnp.zeros_like(acc_ref)

    acc_ref[...] += lax.dot_general(
        c_ref[...], t_ref[...],
        dimension_numbers=(((1,), (1,)), ((), ())),
        preferred_element_type=jnp.float32)

    @pl.when(k == NSTEPS - 1)
    def _():
        s = acc_ref[...]
        nf = n_ref[...].astype(jnp.float32)
        x = s / nf[:, None]
        mean = jnp.mean(x, axis=-1, keepdims=True)
        var = jnp.mean((x - mean) ** 2, axis=-1, keepdims=True)
        xn = (x - mean) * lax.rsqrt(var + 1e-5)
        xn = xn * gamma_ref[...][None, :] + beta_ref[...][None, :]
        logits = jnp.dot(xn, w_ref[...],
                         preferred_element_type=jnp.float32)
        logits = logits + b_ref[...][None, :]
        out_ref[...] = jax.nn.sigmoid(logits)[:, 0]


_tc_matmul = pl.pallas_call(
    _tc_matmul_kernel,
    grid=(NSTEPS,),
    in_specs=[
        pl.BlockSpec((B, KB), lambda k: (0, k)),
        pl.BlockSpec((P, KB), lambda k: (0, k)),
        pl.BlockSpec((B,), lambda k: (0,)),
        pl.BlockSpec((P,), lambda k: (0,)),
        pl.BlockSpec((P,), lambda k: (0,)),
        pl.BlockSpec((P, 1), lambda k: (0, 0)),
        pl.BlockSpec((1,), lambda k: (0,)),
    ],
    out_specs=pl.BlockSpec((B,), lambda k: (0,)),
    out_shape=jax.ShapeDtypeStruct((B,), jnp.float32),
    scratch_shapes=[pltpu.VMEM((B, P), jnp.float32)],
)


def kernel(X, N, table, gamma, beta, W, b):
    X = X.astype(jnp.int32)
    N = N.astype(jnp.int32)
    counts = _sc_hist(X, N)
    table_t = jnp.swapaxes(table, 0, 1)
    return _tc_matmul(counts, table_t, N, gamma, beta, W, b)
